# baseline (speedup 1.0000x reference)
import functools

import jax
import jax.numpy as jnp
from jax import lax
from jax.experimental import pallas as pl
from jax.experimental.pallas import tpu as pltpu
from jax.experimental.pallas import tpu_sc as plsc

_B, _N, _D, _K = 4, 10000, 256, 6
_H = 4 * _D
_NPAD = 10240
_RTILE = 320
_TPB = _NPAD // _RTILE
_NW = 32

_HIMASK = -65536
_i32 = functools.partial(lax.bitcast_convert_type, new_dtype=jnp.int32)
_f32 = functools.partial(lax.bitcast_convert_type, new_dtype=jnp.float32)


def _make_gather6_sum(pw, chunk):
    rw = _NPAD // _NW
    nchunk = rw // chunk
    assert nchunk % 2 == 0 and nchunk >= 4
    mesh = plsc.VectorSubcoreMesh(core_axis_name="c", subcore_axis_name="s")

    @functools.partial(
        pl.kernel,
        mesh=mesh,
        out_type=jax.ShapeDtypeStruct((_NPAD, pw), jnp.float32),
        scratch_types=[
            pltpu.VMEM((_K * rw,), jnp.int32),
            pltpu.VMEM((chunk * _K, pw), jnp.float32),
            pltpu.VMEM((chunk * _K, pw), jnp.float32),
            pltpu.VMEM((chunk, pw), jnp.float32),
            pltpu.SemaphoreType.DMA,
            pltpu.SemaphoreType.DMA,
        ],
    )
    def gather6(table_hbm, idx_hbm, out_hbm, idx_v, bufa_v, bufb_v, acc_v,
                sema, semb):
        wid = lax.axis_index("s") * 2 + lax.axis_index("c")
        base = wid * rw
        pltpu.sync_copy(idx_hbm.at[pl.ds(base * _K, rw * _K)], idx_v)

        def copies(c, bufs, sem):
            return [pltpu.make_async_copy(
                table_hbm.at[idx_v.at[pl.ds(c * chunk * _K, chunk * _K)]],
                bufs, sem)]

        def fire(c, bufs, sem):
            for cp in copies(c, bufs, sem):
                cp.start()

        def drain(c, bufs, sem):
            for cp in copies(c, bufs, sem):
                cp.wait()

        def process(c, bufs):
            sixteen = 16

            def col_body(j, jcarry):
                s = pl.ds(pl.multiple_of(j * 16, 16), 16)
                for r in range(chunk):
                    sa = None
                    sb = None
                    for kk in range(_K):
                        vi = _i32(bufs[r * _K + kk, s])
                        lo = _f32(lax.shift_left(vi, sixteen))
                        hi = _f32(vi & _HIMASK)
                        sa = lo if sa is None else sa + lo
                        sb = hi if sb is None else sb + hi
                    packed = ((_i32(sb) & _HIMASK)
                              | lax.shift_right_logical(_i32(sa), sixteen))
                    acc_v[r, s] = _f32(packed)
                return jcarry

            lax.fori_loop(0, pw // 16, col_body, 0)
            pltpu.sync_copy(acc_v, out_hbm.at[pl.ds(base + c * chunk, chunk)])

        fire(0, bufa_v, sema)

        def pair_body(i, carry):
            c0 = i * 2
            fire(c0 + 1, bufb_v, semb)
            drain(c0, bufa_v, sema)
            process(c0, bufa_v)
            fire(c0 + 2, bufa_v, sema)
            drain(c0 + 1, bufb_v, semb)
            process(c0 + 1, bufb_v)
            return carry

        lax.fori_loop(0, nchunk // 2 - 1, pair_body, 0)
        c0 = nchunk - 2
        fire(c0 + 1, bufb_v, semb)
        drain(c0, bufa_v, sema)
        process(c0, bufa_v)
        drain(c0 + 1, bufb_v, semb)
        process(c0 + 1, bufb_v)

    return gather6


def _gelu(x):
    return 0.5 * x * (1.0 + lax.erf(x * 0.7071067811865476))


def _ln(x, g, b):
    mu = jnp.mean(x, axis=-1, keepdims=True)
    xc = x - mu
    var = jnp.mean(xc * xc, axis=-1, keepdims=True)
    return xc * lax.rsqrt(var + 1e-5) * g + b


def _unpk(p):
    pi = _i32(p)
    lo = _f32(lax.shift_left(pi, 16))
    hi = _f32(pi & _HIMASK)
    return jnp.concatenate([lo, hi], axis=1)


def _pk(x):
    lo = _i32(x[:, :128])
    hi = _i32(x[:, 128:])
    return _f32((hi & _HIMASK) | lax.shift_right_logical(lo, 16))


def _proj_body(lat_r, wg_r, bg_r, g_r):
    g_r[...] = (jnp.dot(lat_r[...], wg_r[...],
                        preferred_element_type=jnp.float32) + bg_r[...])


def _blk0_body(pos_r, psum_r, g_r, lng_r, lnb_r, w1_r, b1_r, w2_r, b2_r,
               h0_r):
    g = g_r[pl.ds(pl.program_id(0), 1), :]
    agg = g + (pos_r[...] + _unpk(psum_r[...])) * (1.0 / 7.0)
    h = _ln(agg, lng_r[...], lnb_r[...])
    h = _gelu(
        jnp.dot(h.astype(jnp.bfloat16), w1_r[...],
                preferred_element_type=jnp.float32) + b1_r[...])
    h0 = (jnp.dot(h.astype(jnp.bfloat16), w2_r[...],
                  preferred_element_type=jnp.float32) + b2_r[...])
    h0_r[...] = _pk(h0)


def _blk1_body(pos_r, psum_r, h0_r, hsum_r, g_r, lng_r, lnb_r, w1_r, b1_r,
               w2_r, b2_r, lnhg_r, lnhb_r, wh_r, bh_r, out_r):
    g = g_r[pl.ds(pl.program_id(0), 1), :]
    pos = pos_r[...]
    h0 = _unpk(h0_r[...])
    agg = g + (pos + _unpk(psum_r[...]) + h0
               + _unpk(hsum_r[...])) * (1.0 / 7.0)
    h = _ln(agg, lng_r[...], lnb_r[...])
    h = _gelu(
        jnp.dot(h.astype(jnp.bfloat16), w1_r[...],
                preferred_element_type=jnp.float32) + b1_r[...])
    h1 = (jnp.dot(h.astype(jnp.bfloat16), w2_r[...],
                  preferred_element_type=jnp.float32) + b2_r[...])
    nodes2 = g + pos + h0 + h1
    hn = _ln(nodes2, lnhg_r[...], lnhb_r[...])
    out_r[...] = (jnp.dot(hn, wh_r[...], preferred_element_type=jnp.float32)
                  + bh_r[...])


_gather_pos = _make_gather6_sum(_D // 2, 16)
_gather_h0 = _make_gather6_sum(_B * _D // 2, 16)


def kernel(latent_token, pos_embed, Wg, bg, ln0_g, ln0_b, W1_0, b1_0, W2_0,
           b2_0, ln1_g, ln1_b, W1_1, b1_1, W2_1, b2_1, lnh_g, lnh_b, Wh, bh,
           adj_matrix):
    pos_p = jnp.pad(pos_embed[0], ((0, _NPAD - _N), (0, 0)))
    adj_p = jnp.pad(adj_matrix, ((0, _NPAD - _N), (0, 0)))
    idx_a = adj_p.reshape(-1)
    lat_p = jnp.pad(latent_token, ((0, 8 - _B), (0, 0)))
    wh_p = jnp.pad(Wh, ((0, 0), (0, 6)))
    bh_p = jnp.pad(bh, (0, 6))

    pos16 = pos_p.astype(jnp.bfloat16)
    lo16 = lax.bitcast_convert_type(pos16[:, :128], jnp.uint16)
    hi16 = lax.bitcast_convert_type(pos16[:, 128:], jnp.uint16)
    pos_packed = _f32(lax.shift_left(hi16.astype(jnp.int32), 16)
                      | lo16.astype(jnp.int32))

    g8 = pl.pallas_call(
        _proj_body,
        out_shape=jax.ShapeDtypeStruct((8, _D), jnp.float32),
    )(lat_p, Wg, bg)

    psum = _gather_pos(pos_packed, idx_a)

    full = lambda shape: pl.BlockSpec(shape, lambda b, t: (0,) * len(shape))
    row_in = pl.BlockSpec((_RTILE, _D), lambda b, t: (t, 0))
    pk_in = pl.BlockSpec((_RTILE, _D // 2), lambda b, t: (t, 0))
    pk_tb = pl.BlockSpec((_RTILE, _D // 2), lambda b, t: (t, b))
    g_spec = pl.BlockSpec((8, _D), lambda b, t: (0, 0))

    h0 = pl.pallas_call(
        _blk0_body,
        grid=(_B, _TPB),
        in_specs=[row_in, pk_in, g_spec, full((_D,)), full((_D,)),
                  full((_D, _H)), full((_H,)), full((_H, _D)), full((_D,))],
        out_specs=pk_tb,
        out_shape=jax.ShapeDtypeStruct((_NPAD, _B * _D // 2), jnp.float32),
    )(pos_p, psum, g8, ln0_g, ln0_b, W1_0.astype(jnp.bfloat16), b1_0,
      W2_0.astype(jnp.bfloat16), b2_0)

    hsum = _gather_h0(h0, idx_a)

    out = pl.pallas_call(
        _blk1_body,
        grid=(_B, _TPB),
        in_specs=[row_in, pk_in, pk_tb, pk_tb, g_spec,
                  full((_D,)), full((_D,)), full((_D, _H)), full((_H,)),
                  full((_H, _D)), full((_D,)), full((_D,)), full((_D,)),
                  full((_D, 8)), full((8,))],
        out_specs=pl.BlockSpec((_RTILE, 8), lambda b, t: (b * _TPB + t, 0)),
        out_shape=jax.ShapeDtypeStruct((_B * _NPAD, 8), jnp.float32),
    )(pos_p, psum, h0, hsum, g8, ln1_g, ln1_b, W1_1.astype(jnp.bfloat16),
      b1_1, W2_1.astype(jnp.bfloat16), b2_1, lnh_g, lnh_b, wh_p, bh_p)

    out = out.reshape(_B, _NPAD, 8)[:, :_N, :2]
    return jnp.transpose(out, (0, 2, 1))


# speedup vs baseline: 1.1087x; 1.1087x over previous
import functools

import jax
import jax.numpy as jnp
from jax import lax
from jax.experimental import pallas as pl
from jax.experimental.pallas import tpu as pltpu
from jax.experimental.pallas import tpu_sc as plsc

_B, _N, _D, _K = 4, 10000, 256, 6
_H = 4 * _D
_NPAD = 10240
_RTILE = 640
_TPB = _NPAD // _RTILE
_NW = 32

_HIMASK = -65536
_i32 = functools.partial(lax.bitcast_convert_type, new_dtype=jnp.int32)
_f32 = functools.partial(lax.bitcast_convert_type, new_dtype=jnp.float32)


def _make_gather6_sum(pw, chunk):
    rw = _NPAD // _NW
    nchunk = rw // chunk
    assert nchunk % 2 == 0 and nchunk >= 4
    mesh = plsc.VectorSubcoreMesh(core_axis_name="c", subcore_axis_name="s")

    @functools.partial(
        pl.kernel,
        mesh=mesh,
        out_type=jax.ShapeDtypeStruct((_NPAD, pw), jnp.float32),
        scratch_types=[
            pltpu.VMEM((_K * rw,), jnp.int32),
            pltpu.VMEM((chunk * _K, pw), jnp.float32),
            pltpu.VMEM((chunk * _K, pw), jnp.float32),
            pltpu.VMEM((chunk, pw), jnp.float32),
            pltpu.SemaphoreType.DMA,
            pltpu.SemaphoreType.DMA,
        ],
    )
    def gather6(table_hbm, idx_hbm, out_hbm, idx_v, bufa_v, bufb_v, acc_v,
                sema, semb):
        wid = lax.axis_index("s") * 2 + lax.axis_index("c")
        base = wid * rw
        pltpu.sync_copy(idx_hbm.at[pl.ds(base * _K, rw * _K)], idx_v)

        def copies(c, bufs, sem):
            return [pltpu.make_async_copy(
                table_hbm.at[idx_v.at[pl.ds(c * chunk * _K, chunk * _K)]],
                bufs, sem)]

        def fire(c, bufs, sem):
            for cp in copies(c, bufs, sem):
                cp.start()

        def drain(c, bufs, sem):
            for cp in copies(c, bufs, sem):
                cp.wait()

        def process(c, bufs):
            sixteen = 16

            def col_body(j, jcarry):
                s = pl.ds(pl.multiple_of(j * 16, 16), 16)
                for r in range(chunk):
                    sa = None
                    sb = None
                    for kk in range(_K):
                        vi = _i32(bufs[r * _K + kk, s])
                        lo = _f32(lax.shift_left(vi, sixteen))
                        hi = _f32(vi & _HIMASK)
                        sa = lo if sa is None else sa + lo
                        sb = hi if sb is None else sb + hi
                    packed = ((_i32(sb) & _HIMASK)
                              | lax.shift_right_logical(_i32(sa), sixteen))
                    acc_v[r, s] = _f32(packed)
                return jcarry

            lax.fori_loop(0, pw // 16, col_body, 0)
            pltpu.sync_copy(acc_v, out_hbm.at[pl.ds(base + c * chunk, chunk)])

        fire(0, bufa_v, sema)

        def pair_body(i, carry):
            c0 = i * 2
            fire(c0 + 1, bufb_v, semb)
            drain(c0, bufa_v, sema)
            process(c0, bufa_v)
            fire(c0 + 2, bufa_v, sema)
            drain(c0 + 1, bufb_v, semb)
            process(c0 + 1, bufb_v)
            return carry

        lax.fori_loop(0, nchunk // 2 - 1, pair_body, 0)
        c0 = nchunk - 2
        fire(c0 + 1, bufb_v, semb)
        drain(c0, bufa_v, sema)
        process(c0, bufa_v)
        drain(c0 + 1, bufb_v, semb)
        process(c0 + 1, bufb_v)

    return gather6


def _gelu(x):
    return 0.5 * x * (1.0 + lax.erf(x * 0.7071067811865476))


def _ln_nb(x, eps):
    mu = jnp.mean(x, axis=-1, keepdims=True)
    xc = x - mu
    var = jnp.mean(xc * xc, axis=-1, keepdims=True)
    return xc * lax.rsqrt(var + eps)


def _unpk(p):
    pi = _i32(p)
    lo = _f32(lax.shift_left(pi, 16))
    hi = _f32(pi & _HIMASK)
    return jnp.concatenate([lo, hi], axis=1)


def _pk(x):
    lo = _i32(x[:, :128])
    hi = _i32(x[:, 128:])
    return _f32((hi & _HIMASK) | lax.shift_right_logical(lo, 16))


def _latent_g(lat_r, wg_r, bg_r):
    g_all = (jnp.dot(lat_r[...], wg_r[...],
                     preferred_element_type=jnp.float32) + bg_r[...])
    rows = lax.broadcasted_iota(jnp.int32, (8, 1), 0)
    sel = jnp.where(rows == pl.program_id(0), g_all, 0.0)
    return jnp.sum(sel, axis=0, keepdims=True)


def _blk0_body(pos_r, psum_r, lat_r, wg_r, bg_r, w1_r, b1_r, w2_r, b2_r,
               h0_r):
    g = _latent_g(lat_r, wg_r, bg_r)
    agg7 = 7.0 * g + pos_r[...] + _unpk(psum_r[...])
    h = _ln_nb(agg7, 49e-5)
    h = _gelu(
        jnp.dot(h.astype(jnp.bfloat16), w1_r[...],
                preferred_element_type=jnp.float32) + b1_r[...])
    h0 = (jnp.dot(h.astype(jnp.bfloat16), w2_r[...],
                  preferred_element_type=jnp.float32) + b2_r[...])
    h0_r[...] = _pk(h0)


def _blk1_body(pos_r, psum_r, h0_r, hsum_r, lat_r, wg_r, bg_r, w1_r, b1_r,
               w2_r, b2_r, wh_r, bh_r, out_r):
    g = _latent_g(lat_r, wg_r, bg_r)
    pos = pos_r[...]
    h0 = _unpk(h0_r[...])
    agg7 = 7.0 * g + pos + _unpk(psum_r[...]) + h0 + _unpk(hsum_r[...])
    h = _ln_nb(agg7, 49e-5)
    h = _gelu(
        jnp.dot(h.astype(jnp.bfloat16), w1_r[...],
                preferred_element_type=jnp.float32) + b1_r[...])
    h1 = (jnp.dot(h.astype(jnp.bfloat16), w2_r[...],
                  preferred_element_type=jnp.float32) + b2_r[...])
    nodes2 = g + pos + h0 + h1
    hn = _ln_nb(nodes2, 1e-5)
    out_r[...] = (jnp.dot(hn, wh_r[...], preferred_element_type=jnp.float32)
                  + bh_r[...])


_gather_pos = _make_gather6_sum(_D // 2, 16)
_gather_h0 = _make_gather6_sum(_B * _D // 2, 16)


def kernel(latent_token, pos_embed, Wg, bg, ln0_g, ln0_b, W1_0, b1_0, W2_0,
           b2_0, ln1_g, ln1_b, W1_1, b1_1, W2_1, b2_1, lnh_g, lnh_b, Wh, bh,
           adj_matrix):
    pos_p = jnp.pad(pos_embed[0], ((0, _NPAD - _N), (0, 0)))
    adj_p = jnp.pad(adj_matrix, ((0, _NPAD - _N), (0, 0)))
    idx_a = adj_p.reshape(-1)
    lat_p = jnp.pad(latent_token, ((0, 8 - _B), (0, 0)))
    wh_p = jnp.pad(Wh, ((0, 0), (0, 6)))
    bh_p = jnp.pad(bh, (0, 6))

    pos16 = pos_p.astype(jnp.bfloat16)
    lo16 = lax.bitcast_convert_type(pos16[:, :128], jnp.uint16)
    hi16 = lax.bitcast_convert_type(pos16[:, 128:], jnp.uint16)
    pos_packed = _f32(lax.shift_left(hi16.astype(jnp.int32), 16)
                      | lo16.astype(jnp.int32))

    w1_0f = (ln0_g[:, None] * W1_0).astype(jnp.bfloat16)
    b1_0f = b1_0 + ln0_b @ W1_0
    w1_1f = (ln1_g[:, None] * W1_1).astype(jnp.bfloat16)
    b1_1f = b1_1 + ln1_b @ W1_1
    whf = lnh_g[:, None] * wh_p
    bhf = bh_p + lnh_b @ wh_p

    psum = _gather_pos(pos_packed, idx_a)

    full = lambda shape: pl.BlockSpec(shape, lambda b, t: (0,) * len(shape))
    row_in = pl.BlockSpec((_RTILE, _D), lambda b, t: (t, 0))
    pk_in = pl.BlockSpec((_RTILE, _D // 2), lambda b, t: (t, 0))
    pk_tb = pl.BlockSpec((_RTILE, _D // 2), lambda b, t: (t, b))
    lat_spec = pl.BlockSpec((8, _D), lambda b, t: (0, 0))

    h0 = pl.pallas_call(
        _blk0_body,
        grid=(_B, _TPB),
        in_specs=[row_in, pk_in, lat_spec, full((_D, _D)), full((_D,)),
                  full((_D, _H)), full((_H,)), full((_H, _D)), full((_D,))],
        out_specs=pk_tb,
        out_shape=jax.ShapeDtypeStruct((_NPAD, _B * _D // 2), jnp.float32),
    )(pos_p, psum, lat_p, Wg, bg, w1_0f, b1_0f,
      W2_0.astype(jnp.bfloat16), b2_0)

    hsum = _gather_h0(h0, idx_a)

    out = pl.pallas_call(
        _blk1_body,
        grid=(_B, _TPB),
        in_specs=[row_in, pk_in, pk_tb, pk_tb, lat_spec,
                  full((_D, _D)), full((_D,)), full((_D, _H)), full((_H,)),
                  full((_H, _D)), full((_D,)), full((_D, 8)), full((8,))],
        out_specs=pl.BlockSpec((_RTILE, 8), lambda b, t: (b * _TPB + t, 0)),
        out_shape=jax.ShapeDtypeStruct((_B * _NPAD, 8), jnp.float32),
    )(pos_p, psum, h0, hsum, lat_p, Wg, bg, w1_1f, b1_1f,
      W2_1.astype(jnp.bfloat16), b2_1, whf, bhf)

    out = out.reshape(_B, _NPAD, 8)[:, :_N, :2]
    return jnp.transpose(out, (0, 2, 1))
